# initial kernel scaffold (unmeasured)
import jax
import jax.numpy as jnp
from jax import lax
from jax.experimental import pallas as pl
from jax.experimental.pallas import tpu as pltpu


def kernel(
    x,
):
    def body(*refs):
        pass

    out_shape = jax.ShapeDtypeStruct(..., jnp.float32)
    return pl.pallas_call(body, out_shape=out_shape)(...)


# baseline (device time: 29192 ns/iter reference)
import jax
import jax.numpy as jnp
from jax import lax
from jax.experimental import pallas as pl
from jax.experimental.pallas import tpu as pltpu

N_Y = 4


def kernel(x):
    _, m, n_total = x.shape
    n_chunk = n_total // N_Y
    x = x.reshape(m, n_total)

    def body(x_ref, out_ref, send_buf, recv_bufs, send_sem, recv_sems):
        my_x = lax.axis_index("x")
        my_y = lax.axis_index("y")
        my_z = lax.axis_index("z")
        right = (my_y + 1) % N_Y
        left = (my_y + N_Y - 1) % N_Y

        barrier_sem = pltpu.get_barrier_semaphore()
        for nbr in (left, right):
            pl.semaphore_signal(
                barrier_sem, inc=1,
                device_id=(my_x, nbr, my_z),
                device_id_type=pl.DeviceIdType.MESH,
            )
        pl.semaphore_wait(barrier_sem, 2)

        def blk(c):
            return pl.ds(c * n_chunk, n_chunk)

        send_buf[...] = x_ref[:, blk((my_y + N_Y - 1) % N_Y)].astype(jnp.bfloat16)
        for h in range(N_Y - 1):
            rdma = pltpu.make_async_remote_copy(
                src_ref=send_buf,
                dst_ref=recv_bufs.at[h],
                send_sem=send_sem,
                recv_sem=recv_sems.at[h],
                device_id=(my_x, right, my_z),
                device_id_type=pl.DeviceIdType.MESH,
            )
            rdma.start()
            rdma.wait()
            if h < N_Y - 2:
                c = (my_y + N_Y - 2 - h) % N_Y
                send_buf[...] = recv_bufs[h] + x_ref[:, blk(c)].astype(jnp.bfloat16)
        out_ref[...] = recv_bufs[N_Y - 2].astype(jnp.float32) + x_ref[:, blk(my_y)]

    return pl.pallas_call(
        body,
        out_shape=jax.ShapeDtypeStruct((m, n_chunk), jnp.float32),
        in_specs=[pl.BlockSpec(memory_space=pltpu.VMEM)],
        out_specs=pl.BlockSpec(memory_space=pltpu.VMEM),
        scratch_shapes=[
            pltpu.VMEM((m, n_chunk), jnp.bfloat16),
            pltpu.VMEM((N_Y - 1, m, n_chunk), jnp.bfloat16),
            pltpu.SemaphoreType.DMA,
            pltpu.SemaphoreType.DMA((N_Y - 1,)),
        ],
        compiler_params=pltpu.CompilerParams(collective_id=0),
    )(x)


# device time: 28559 ns/iter; 1.0222x vs baseline; 1.0222x over previous
import jax
import jax.numpy as jnp
from jax import lax
from jax.experimental import pallas as pl
from jax.experimental.pallas import tpu as pltpu

N_Y = 4


def kernel(x):
    _, m, n_total = x.shape
    n_chunk = n_total // N_Y
    half = n_chunk // 2
    x = x.reshape(m, n_total)
    bf16 = jnp.bfloat16

    def body(x_ref, out_ref, send_cw, send_ccw, recv_cw, recv_ccw,
             send_sems, recv_sems_cw, recv_sems_ccw):
        my_x = lax.axis_index("x")
        my_y = lax.axis_index("y")
        my_z = lax.axis_index("z")
        right = (my_y + 1) % N_Y
        left = (my_y + N_Y - 1) % N_Y

        barrier_sem = pltpu.get_barrier_semaphore()
        for nbr in (left, right):
            pl.semaphore_signal(
                barrier_sem, inc=1,
                device_id=(my_x, nbr, my_z),
                device_id_type=pl.DeviceIdType.MESH,
            )
        pl.semaphore_wait(barrier_sem, 2)

        def blk_a(c):
            return pl.ds(c * n_chunk, half)

        def blk_b(c):
            return pl.ds(c * n_chunk + half, half)

        send_cw[...] = x_ref[:, blk_a((my_y + N_Y - 1) % N_Y)].astype(bf16)
        send_ccw[...] = x_ref[:, blk_b((my_y + 1) % N_Y)].astype(bf16)
        for h in range(N_Y - 1):
            rdma_cw = pltpu.make_async_remote_copy(
                src_ref=send_cw,
                dst_ref=recv_cw.at[h],
                send_sem=send_sems.at[0],
                recv_sem=recv_sems_cw.at[h],
                device_id=(my_x, right, my_z),
                device_id_type=pl.DeviceIdType.MESH,
            )
            rdma_ccw = pltpu.make_async_remote_copy(
                src_ref=send_ccw,
                dst_ref=recv_ccw.at[h],
                send_sem=send_sems.at[1],
                recv_sem=recv_sems_ccw.at[h],
                device_id=(my_x, left, my_z),
                device_id_type=pl.DeviceIdType.MESH,
            )
            rdma_cw.start()
            rdma_ccw.start()
            rdma_cw.wait()
            if h < N_Y - 2:
                c_cw = (my_y + N_Y - 2 - h) % N_Y
                send_cw[...] = recv_cw[h] + x_ref[:, blk_a(c_cw)].astype(bf16)
            rdma_ccw.wait()
            if h < N_Y - 2:
                c_ccw = (my_y + 2 + h) % N_Y
                send_ccw[...] = recv_ccw[h] + x_ref[:, blk_b(c_ccw)].astype(bf16)
        out_ref[:, :half] = (
            recv_cw[N_Y - 2].astype(jnp.float32) + x_ref[:, blk_a(my_y)]
        )
        out_ref[:, half:] = (
            recv_ccw[N_Y - 2].astype(jnp.float32) + x_ref[:, blk_b(my_y)]
        )

    return pl.pallas_call(
        body,
        out_shape=jax.ShapeDtypeStruct((m, n_chunk), jnp.float32),
        in_specs=[pl.BlockSpec(memory_space=pltpu.VMEM)],
        out_specs=pl.BlockSpec(memory_space=pltpu.VMEM),
        scratch_shapes=[
            pltpu.VMEM((m, half), bf16),
            pltpu.VMEM((m, half), bf16),
            pltpu.VMEM((N_Y - 1, m, half), bf16),
            pltpu.VMEM((N_Y - 1, m, half), bf16),
            pltpu.SemaphoreType.DMA((2,)),
            pltpu.SemaphoreType.DMA((N_Y - 1,)),
            pltpu.SemaphoreType.DMA((N_Y - 1,)),
        ],
        compiler_params=pltpu.CompilerParams(collective_id=0),
    )(x)


# device time: 28378 ns/iter; 1.0287x vs baseline; 1.0064x over previous
import jax
import jax.numpy as jnp
from jax import lax
from jax.experimental import pallas as pl
from jax.experimental.pallas import tpu as pltpu

N_Y = 4


def kernel(x):
    _, m, n_total = x.shape
    n_chunk = n_total // N_Y
    half = n_chunk // 2
    x = x.reshape(m, n_total)
    bf16 = jnp.bfloat16

    def body(x_ref, out_ref, xa, xb, send_cw, send_ccw, recv_cw, recv_ccw,
             send_sems_cw, send_sems_ccw, recv_sems_cw, recv_sems_ccw):
        my_x = lax.axis_index("x")
        my_y = lax.axis_index("y")
        my_z = lax.axis_index("z")
        right = (my_y + 1) % N_Y
        left = (my_y + N_Y - 1) % N_Y

        for c in range(N_Y):
            xa[c] = x_ref[:, c * n_chunk:c * n_chunk + half].astype(bf16)
            xb[c] = x_ref[:, c * n_chunk + half:(c + 1) * n_chunk].astype(bf16)

        send_cw[0] = xa[left]
        send_ccw[0] = xb[right]

        barrier_sem = pltpu.get_barrier_semaphore()
        for nbr in (left, right):
            pl.semaphore_signal(
                barrier_sem, inc=1,
                device_id=(my_x, nbr, my_z),
                device_id_type=pl.DeviceIdType.MESH,
            )
        pl.semaphore_wait(barrier_sem, 2)

        rdmas = []
        for h in range(N_Y - 1):
            rdma_cw = pltpu.make_async_remote_copy(
                src_ref=send_cw.at[h],
                dst_ref=recv_cw.at[h],
                send_sem=send_sems_cw.at[h],
                recv_sem=recv_sems_cw.at[h],
                device_id=(my_x, right, my_z),
                device_id_type=pl.DeviceIdType.MESH,
            )
            rdma_ccw = pltpu.make_async_remote_copy(
                src_ref=send_ccw.at[h],
                dst_ref=recv_ccw.at[h],
                send_sem=send_sems_ccw.at[h],
                recv_sem=recv_sems_ccw.at[h],
                device_id=(my_x, left, my_z),
                device_id_type=pl.DeviceIdType.MESH,
            )
            rdma_cw.start()
            rdma_ccw.start()
            rdmas += [rdma_cw, rdma_ccw]
            rdma_cw.wait_recv()
            if h < N_Y - 2:
                send_cw[h + 1] = recv_cw[h] + xa[(my_y + N_Y - 2 - h) % N_Y]
            rdma_ccw.wait_recv()
            if h < N_Y - 2:
                send_ccw[h + 1] = recv_ccw[h] + xb[(my_y + 2 + h) % N_Y]

        out_ref[:, :half] = (
            recv_cw[N_Y - 2].astype(jnp.float32) + xa[my_y].astype(jnp.float32)
        )
        out_ref[:, half:] = (
            recv_ccw[N_Y - 2].astype(jnp.float32) + xb[my_y].astype(jnp.float32)
        )
        for r in rdmas:
            r.wait_send()

    return pl.pallas_call(
        body,
        out_shape=jax.ShapeDtypeStruct((m, n_chunk), jnp.float32),
        in_specs=[pl.BlockSpec(memory_space=pltpu.VMEM)],
        out_specs=pl.BlockSpec(memory_space=pltpu.VMEM),
        scratch_shapes=[
            pltpu.VMEM((N_Y, m, half), bf16),
            pltpu.VMEM((N_Y, m, half), bf16),
            pltpu.VMEM((N_Y - 1, m, half), bf16),
            pltpu.VMEM((N_Y - 1, m, half), bf16),
            pltpu.VMEM((N_Y - 1, m, half), bf16),
            pltpu.VMEM((N_Y - 1, m, half), bf16),
            pltpu.SemaphoreType.DMA((N_Y - 1,)),
            pltpu.SemaphoreType.DMA((N_Y - 1,)),
            pltpu.SemaphoreType.DMA((N_Y - 1,)),
            pltpu.SemaphoreType.DMA((N_Y - 1,)),
        ],
        compiler_params=pltpu.CompilerParams(collective_id=0),
    )(x)


# device time: 3796 ns/iter; 7.6902x vs baseline; 7.4758x over previous
import jax
import jax.numpy as jnp
from jax import lax
from jax.experimental import pallas as pl
from jax.experimental.pallas import tpu as pltpu

N_Y = 4
MESH = pl.DeviceIdType.MESH


def kernel(x):
    _, m, n_total = x.shape
    n_chunk = n_total // N_Y
    half = n_chunk // 2
    x = x.reshape(m, n_total)
    bf16 = jnp.bfloat16
    f32 = jnp.float32

    def body(x_ref, out_ref, xh, comb1, comb2, outh, rb, rc, rd, re, rx,
             ssems, sx_sem, rb_sems, rc_sem, rd_sem, re_sem, rx_sem):
        p = lax.axis_index("x")
        i = lax.axis_index("y")
        z = lax.axis_index("z")

        for c in range(N_Y):
            xh[c] = x_ref[:, pl.ds(c * n_chunk + p * half, half)].astype(bf16)

        barrier = pltpu.get_barrier_semaphore()

        def sig_y(y_t):
            pl.semaphore_signal(barrier, inc=1, device_id=(p, y_t, z),
                                device_id_type=MESH)

        def sig_x():
            pl.semaphore_signal(barrier, inc=1, device_id=(1 - p, i, z),
                                device_id_type=MESH)

        def rsend(src, dst, ssem, rsem, y_t):
            r = pltpu.make_async_remote_copy(
                src_ref=src, dst_ref=dst, send_sem=ssem, recv_sem=rsem,
                device_id=(p, y_t, z), device_id_type=MESH)
            r.start()
            return r

        def rwait(dst, rsem):
            r = pltpu.make_async_remote_copy(
                src_ref=dst, dst_ref=dst, send_sem=sx_sem, recv_sem=rsem,
                device_id=(p, i, z), device_id_type=MESH)
            r.wait_recv()

        @pl.when(i == 0)
        def _():
            sig_y(1); sig_y(2); sig_x()
            pl.semaphore_wait(barrier, 2)
            r1 = rsend(xh.at[2], rb.at[0], ssems.at[0], rb_sems.at[0], 1)
            r2 = rsend(xh.at[3], rb.at[1], ssems.at[1], rb_sems.at[1], 1)
            r3 = rsend(xh.at[1], rb.at[2], ssems.at[2], rb_sems.at[2], 1)
            rwait(re, re_sem)
            rwait(rc, rc_sem)
            outh[...] = xh[0] + re[...] + rc[...]
            r1.wait_send(); r2.wait_send(); r3.wait_send()

        @pl.when(i == 1)
        def _():
            sig_y(0); sig_y(2); sig_x()
            pl.semaphore_wait(barrier, 4)
            r1 = rsend(xh.at[0], re, ssems.at[0], re_sem, 0)
            rwait(rb.at[0], rb_sems.at[0])
            comb1[...] = rb[0] + xh[2]
            r2 = rsend(comb1, rd, ssems.at[1], rd_sem, 2)
            rwait(rb.at[1], rb_sems.at[1])
            comb2[...] = rb[1] + xh[3]
            r3 = rsend(comb2, rc, ssems.at[2], rc_sem, 3)
            rwait(rb.at[2], rb_sems.at[2])
            rwait(rd, rd_sem)
            outh[...] = xh[1] + rb[2] + rd[...]
            r1.wait_send(); r2.wait_send(); r3.wait_send()

        @pl.when(i == 2)
        def _():
            sig_y(1); sig_y(3); sig_x()
            pl.semaphore_wait(barrier, 4)
            r1 = rsend(xh.at[3], re, ssems.at[0], re_sem, 3)
            rwait(rb.at[0], rb_sems.at[0])
            comb1[...] = rb[0] + xh[1]
            r2 = rsend(comb1, rd, ssems.at[1], rd_sem, 1)
            rwait(rb.at[1], rb_sems.at[1])
            comb2[...] = rb[1] + xh[0]
            r3 = rsend(comb2, rc, ssems.at[2], rc_sem, 0)
            rwait(rb.at[2], rb_sems.at[2])
            rwait(rd, rd_sem)
            outh[...] = xh[2] + rb[2] + rd[...]
            r1.wait_send(); r2.wait_send(); r3.wait_send()

        @pl.when(i == 3)
        def _():
            sig_y(1); sig_y(2); sig_x()
            pl.semaphore_wait(barrier, 2)
            r1 = rsend(xh.at[1], rb.at[0], ssems.at[0], rb_sems.at[0], 2)
            r2 = rsend(xh.at[0], rb.at[1], ssems.at[1], rb_sems.at[1], 2)
            r3 = rsend(xh.at[2], rb.at[2], ssems.at[2], rb_sems.at[2], 2)
            rwait(re, re_sem)
            rwait(rc, rc_sem)
            outh[...] = xh[3] + re[...] + rc[...]
            r1.wait_send(); r2.wait_send(); r3.wait_send()

        rxd = pltpu.make_async_remote_copy(
            src_ref=outh, dst_ref=rx, send_sem=sx_sem, recv_sem=rx_sem,
            device_id=(1 - p, i, z), device_id_type=MESH)
        rxd.start()
        rxd.wait()
        out_ref[:, pl.ds(p * half, half)] = outh[...].astype(f32)
        out_ref[:, pl.ds((1 - p) * half, half)] = rx[...].astype(f32)

    return pl.pallas_call(
        body,
        out_shape=jax.ShapeDtypeStruct((m, n_chunk), f32),
        in_specs=[pl.BlockSpec(memory_space=pltpu.VMEM)],
        out_specs=pl.BlockSpec(memory_space=pltpu.VMEM),
        scratch_shapes=[
            pltpu.VMEM((N_Y, m, half), bf16),
            pltpu.VMEM((m, half), bf16),
            pltpu.VMEM((m, half), bf16),
            pltpu.VMEM((m, half), bf16),
            pltpu.VMEM((3, m, half), bf16),
            pltpu.VMEM((m, half), bf16),
            pltpu.VMEM((m, half), bf16),
            pltpu.VMEM((m, half), bf16),
            pltpu.VMEM((m, half), bf16),
            pltpu.SemaphoreType.DMA((3,)),
            pltpu.SemaphoreType.DMA,
            pltpu.SemaphoreType.DMA((3,)),
            pltpu.SemaphoreType.DMA,
            pltpu.SemaphoreType.DMA,
            pltpu.SemaphoreType.DMA,
            pltpu.SemaphoreType.DMA,
        ],
        compiler_params=pltpu.CompilerParams(collective_id=0),
    )(x)
